# baseline (speedup 1.0000x reference)
import functools

import jax
import jax.numpy as jnp
from jax.experimental import pallas as pl
from jax.experimental.pallas import tpu as pltpu

_NSLOTS = 4
_NOUT = 4


def _se_manual(x_hbm, w1_ref, w2_ref, o_hbm, ibuf, obuf, in_sem, out_sem):
    B, C, S = x_hbm.shape
    inv_S = 1.0 / float(S)
    w1 = w1_ref[...]
    w2 = w2_ref[...]

    def start_in(b):
        pltpu.make_async_copy(
            x_hbm.at[b], ibuf.at[b % _NSLOTS], in_sem.at[b % _NSLOTS]
        ).start()

    def wait_in(b):
        pltpu.make_async_copy(
            x_hbm.at[0], ibuf.at[b % _NSLOTS], in_sem.at[b % _NSLOTS]
        ).wait()

    def start_out(b):
        pltpu.make_async_copy(
            obuf.at[b % _NOUT], o_hbm.at[b], out_sem.at[b % _NOUT]
        ).start()

    def wait_out(b):
        pltpu.make_async_copy(
            obuf.at[b % _NOUT], o_hbm.at[0], out_sem.at[b % _NOUT]
        ).wait()

    for b in range(_NSLOTS):
        start_in(b)
    for b in range(B):
        wait_in(b)
        xb = ibuf[b % _NSLOTS]
        mean = jnp.sum(xb, axis=-1, keepdims=True,
                       dtype=jnp.float32) * inv_S
        h = jnp.maximum(
            jnp.dot(w1, mean, preferred_element_type=jnp.float32), 0.0)
        g = jnp.dot(w2, h, preferred_element_type=jnp.float32)
        gate = 1.0 / (1.0 + jnp.exp(-g))
        if b >= _NOUT:
            wait_out(b - _NOUT)
        obuf[b % _NOUT] = xb * gate
        start_out(b)
        if b + _NSLOTS < B:
            start_in(b + _NSLOTS)
    for b in range(B - _NOUT, B):
        wait_out(b)


@jax.jit
def _se3d(x, w1, w2):
    B, C, D, H, W = x.shape
    S = D * H * W
    x3 = x.reshape(B, C, S)
    out = pl.pallas_call(
        _se_manual,
        out_shape=jax.ShapeDtypeStruct((B, C, S), x.dtype),
        in_specs=[
            pl.BlockSpec(memory_space=pltpu.MemorySpace.HBM),
            pl.BlockSpec(memory_space=pltpu.MemorySpace.VMEM),
            pl.BlockSpec(memory_space=pltpu.MemorySpace.VMEM),
        ],
        out_specs=pl.BlockSpec(memory_space=pltpu.MemorySpace.HBM),
        scratch_shapes=[
            pltpu.VMEM((_NSLOTS, 256, 4096), jnp.float32),
            pltpu.VMEM((_NOUT, 256, 4096), jnp.float32),
            pltpu.SemaphoreType.DMA((_NSLOTS,)),
            pltpu.SemaphoreType.DMA((_NOUT,)),
        ],
        compiler_params=pltpu.CompilerParams(
            vmem_limit_bytes=48 * 1024 * 1024,
        ),
    )(x3, w1, w2)
    return out.reshape(B, C, D, H, W)


def kernel(x, w1, w2):
    return _se3d(x, w1, w2)
